# baseline (speedup 1.0000x reference)
import functools
import math

import jax
import jax.numpy as jnp
from jax import lax
from jax.experimental import pallas as pl
from jax.experimental.pallas import tpu as pltpu
from jax.experimental.pallas import tpu_sc as plsc

N = 10000
E = 160000
NC = 2
NS = 16
NW = NC * NS
CH = 125
CPT = E // (NS * CH)
NPAD = 10240
RPT = NPAD // NS


def _make_segsum(d):
  dh = d // 2
  mesh = plsc.VectorSubcoreMesh(core_axis_name="c", subcore_axis_name="s")
  nbuf = 4

  @functools.partial(
      pl.kernel,
      mesh=mesh,
      compiler_params=pltpu.CompilerParams(use_tc_tiling_on_sc=False),
      out_type=jax.ShapeDtypeStruct((N, d), jnp.float32),
      scratch_types=[
          pltpu.VMEM((CPT, CH), jnp.int32),
          pltpu.VMEM((CPT, CH), jnp.int32),
          pltpu.VMEM((nbuf, CH, dh), jnp.float32),
          pltpu.VMEM_SHARED((NPAD, dh), jnp.float32),
          pltpu.SemaphoreType.DMA,
      ],
  )
  def seg(ga_hbm, gb_hbm, srcr_hbm, dstr_hbm, zeros_hbm, out_hbm,
          src_v, dst_v, rows_v, acc_sh, sem):
    c = lax.axis_index("c")
    s = lax.axis_index("s")
    zc = pltpu.async_copy(zeros_hbm.at[pl.ds(s * RPT, RPT)],
                          acc_sh.at[pl.ds(s * RPT, RPT)], sem)
    ic = pltpu.async_copy(srcr_hbm.at[pl.ds(s * CPT, CPT)], src_v, sem)
    jc = pltpu.async_copy(dstr_hbm.at[pl.ds(s * CPT, CPT)], dst_v, sem)
    zc.wait()
    ic.wait()
    jc.wait()
    plsc.subcore_barrier()

    def run(g_hbm):
      for k in range(nbuf - 1):
        pltpu.async_copy(g_hbm.at[src_v.at[k]], rows_v.at[k], sem)

      def body(j, carry):
        @pl.when(j + nbuf - 1 < CPT)
        def _():
          pltpu.async_copy(g_hbm.at[src_v.at[j + nbuf - 1]],
                           rows_v.at[(j + nbuf - 1) % nbuf], sem)
        pltpu.make_async_copy(g_hbm.at[src_v.at[j]],
                              rows_v.at[j % nbuf], sem).wait()
        pltpu.sync_copy(rows_v.at[j % nbuf], acc_sh.at[dst_v.at[j]],
                        add=True)
        return carry

      lax.fori_loop(0, CPT, body, 0)

    @pl.when(c == 0)
    def _():
      run(ga_hbm)

    @pl.when(c == 1)
    def _():
      run(gb_hbm)

    plsc.subcore_barrier()

    @pl.when(s < NS - 1)
    def _():
      pltpu.sync_copy(acc_sh.at[pl.ds(s * RPT, RPT)],
                      out_hbm.at[pl.ds(s * RPT, RPT), pl.ds(c * dh, dh)])

    @pl.when(s == NS - 1)
    def _():
      pltpu.sync_copy(acc_sh.at[pl.ds((NS - 1) * RPT, N - (NS - 1) * RPT)],
                      out_hbm.at[pl.ds((NS - 1) * RPT, N - (NS - 1) * RPT),
                                 pl.ds(c * dh, dh)])

  return seg


_BM = 1000


def _tc_fused(A, Wa, bias, B=None, Wb=None, P=None, Pw=None,
              temb=None, Wt=None, bt=None, relu=True):
  dO = Wa.shape[1] if Wa is not None else A.shape[1]
  grid = (N // _BM,)

  in_specs = [pl.BlockSpec((_BM, A.shape[1]), lambda i: (i, 0))]
  args = [A]
  if Wa is not None:
    in_specs += [pl.BlockSpec(Wa.shape, lambda i: (0, 0))]
    args += [Wa]
  in_specs += [pl.BlockSpec((1, dO), lambda i: (0, 0))]
  args += [bias.reshape(1, dO)]
  if B is not None:
    in_specs += [pl.BlockSpec((_BM, B.shape[1]), lambda i: (i, 0)),
                 pl.BlockSpec(Wb.shape, lambda i: (0, 0))]
    args += [B, Wb]
  if P is not None:
    in_specs += [pl.BlockSpec((_BM, P.shape[1]), lambda i: (i, 0))]
    args += [P]
    if Pw is not None:
      in_specs += [pl.BlockSpec(Pw.shape, lambda i: (0, 0))]
      args += [Pw]
  if temb is not None:
    in_specs += [pl.BlockSpec(temb.shape, lambda i: (0, 0)),
                 pl.BlockSpec(Wt.shape, lambda i: (0, 0)),
                 pl.BlockSpec((1, dO), lambda i: (0, 0))]
    args += [temb, Wt, bt.reshape(1, dO)]

  has_Wa = Wa is not None
  has_B = B is not None
  has_P = P is not None
  has_Pw = Pw is not None
  has_t = temb is not None

  def body(*refs):
    it = iter(refs)
    a_ref = next(it)
    if has_Wa:
      wa_ref = next(it)
    b_ref = next(it)
    if has_Wa:
      acc = jnp.dot(a_ref[...], wa_ref[...],
                    preferred_element_type=jnp.float32)
    else:
      acc = a_ref[...]
    if has_B:
      bm_ref = next(it); wb_ref = next(it)
      acc += jnp.dot(bm_ref[...], wb_ref[...],
                     preferred_element_type=jnp.float32)
    if has_P:
      p_ref = next(it)
      psum = p_ref[...]
      if has_Pw:
        pw_ref = next(it)
        acc += jnp.dot(psum, pw_ref[...], preferred_element_type=jnp.float32)
      else:
        acc += psum
    acc += b_ref[...]
    if relu:
      acc = jnp.maximum(acc, 0.0)
    if has_t:
      t_ref = next(it); wt_ref = next(it); bt_ref = next(it)
      tv = jnp.dot(t_ref[...], wt_ref[...],
                   preferred_element_type=jnp.float32)
      acc += tv[0:1, :] + bt_ref[...]
    out_ref = next(it)
    out_ref[...] = acc

  return pl.pallas_call(
      body,
      grid=grid,
      in_specs=in_specs,
      out_specs=pl.BlockSpec((_BM, dO), lambda i: (i, 0)),
      out_shape=jax.ShapeDtypeStruct((N, dO), jnp.float32),
  )(*args)


def _row(v):
  return v.reshape(1, -1)


def _psum(p_ref):
  return p_ref[...]


def _wspec(W):
  return pl.BlockSpec(W.shape, lambda i: tuple(0 for _ in W.shape))


def _mspec(arr):
  return pl.BlockSpec((_BM, arr.shape[1]), lambda i: (i, 0))


def _pspec(P):
  return pl.BlockSpec((_BM, P.shape[1]), lambda i: (i, 0))


def _oshape(dO):
  return jax.ShapeDtypeStruct((N, dO), jnp.float32)


def _post3_g4(h3, P, Wn3, b3, temb_p, Wt, bt, Wn4):
  def body(h_ref, p_ref, wn3_ref, b3_ref, t_ref, wt_ref, bt_ref, wn4_ref,
           e3_ref, g4_ref):
    acc = h_ref[...] + jnp.dot(_psum(p_ref), wn3_ref[...],
                               preferred_element_type=jnp.float32)
    acc = jnp.maximum(acc + b3_ref[...], 0.0)
    tv = jnp.dot(t_ref[...], wt_ref[...], preferred_element_type=jnp.float32)
    e3 = acc + tv[0:1, :] + bt_ref[...]
    e3_ref[...] = e3
    g4_ref[...] = jnp.dot(e3, wn4_ref[...], preferred_element_type=jnp.float32)

  return pl.pallas_call(
      body, grid=(N // _BM,),
      in_specs=[_mspec(h3), _pspec(P), _wspec(Wn3), _wspec(_row(b3)),
                _wspec(temb_p), _wspec(Wt), _wspec(_row(bt)), _wspec(Wn4)],
      out_specs=[_mspec(h3), pl.BlockSpec((_BM, 128), lambda i: (i, 0))],
      out_shape=[_oshape(256), _oshape(128)],
  )(h3, P, Wn3, _row(b3), temb_p, Wt, _row(bt), Wn4)


def _post4_d4c_g5(h4, P4, b4, Wc4a, e2, Wc4b, bc4, Wn5):
  def body(h_ref, p_ref, b4_ref, wa_ref, e2_ref, wb_ref, bc_ref, wn5_ref,
           d4_ref, g5_ref):
    tmp = jnp.maximum(h_ref[...] + _psum(p_ref) + b4_ref[...], 0.0)
    acc = (jnp.dot(tmp, wa_ref[...], preferred_element_type=jnp.float32)
           + jnp.dot(e2_ref[...], wb_ref[...],
                     preferred_element_type=jnp.float32))
    d4c = jnp.maximum(acc + bc_ref[...], 0.0)
    d4_ref[...] = d4c
    g5_ref[...] = jnp.dot(d4c, wn5_ref[...],
                          preferred_element_type=jnp.float32)

  return pl.pallas_call(
      body, grid=(N // _BM,),
      in_specs=[_mspec(h4), _pspec(P4), _wspec(_row(b4)), _wspec(Wc4a),
                _mspec(e2), _wspec(Wc4b), _wspec(_row(bc4)), _wspec(Wn5)],
      out_specs=[_mspec(h4), pl.BlockSpec((_BM, 96), lambda i: (i, 0))],
      out_shape=[_oshape(128), _oshape(96)],
  )(h4, P4, _row(b4), Wc4a, e2, Wc4b, _row(bc4), Wn5)


def _post5_d5c(h5, P5, b5, Wc5a, e1, Wc5b, bc5):
  def body(h_ref, p_ref, b5_ref, wa_ref, e1_ref, wb_ref, bc_ref, d5_ref):
    tmp = jnp.maximum(h_ref[...] + _psum(p_ref) + b5_ref[...], 0.0)
    acc = (jnp.dot(tmp, wa_ref[...], preferred_element_type=jnp.float32)
           + jnp.dot(e1_ref[...], wb_ref[...],
                     preferred_element_type=jnp.float32))
    d5_ref[...] = jnp.maximum(acc + bc_ref[...], 0.0)

  return pl.pallas_call(
      body, grid=(N // _BM,),
      in_specs=[_mspec(h5), _pspec(P5), _wspec(_row(b5)), _wspec(Wc5a),
                _mspec(e1), _wspec(Wc5b), _wspec(_row(bc5))],
      out_specs=_mspec(h5),
      out_shape=_oshape(96),
  )(h5, P5, _row(b5), Wc5a, e1, Wc5b, _row(bc5))


def _post6_final(h6, P6, Wn6, b6, Wc6a, e0, Wc6b, bc6, Wf, bf):
  def body(h_ref, p_ref, wn6_ref, b6_ref, wa_ref, e0_ref, wb_ref, bc_ref,
           wf_ref, bf_ref, o_ref):
    tmp = h_ref[...] + jnp.dot(_psum(p_ref), wn6_ref[...],
                               preferred_element_type=jnp.float32)
    tmp = jnp.maximum(tmp + b6_ref[...], 0.0)
    acc = (jnp.dot(tmp, wa_ref[...], preferred_element_type=jnp.float32)
           + jnp.dot(e0_ref[...], wb_ref[...],
                     preferred_element_type=jnp.float32))
    u = jnp.maximum(acc + bc_ref[...], 0.0)
    o_ref[...] = jnp.dot(u, wf_ref[...],
                         preferred_element_type=jnp.float32) + bf_ref[...]

  return pl.pallas_call(
      body, grid=(N // _BM,),
      in_specs=[_mspec(h6), _pspec(P6), _wspec(Wn6), _wspec(_row(b6)),
                _wspec(Wc6a), _mspec(e0), _wspec(Wc6b), _wspec(_row(bc6)),
                _wspec(Wf), _wspec(_row(bf))],
      out_specs=pl.BlockSpec((_BM, 256), lambda i: (i, 0)),
      out_shape=_oshape(256),
  )(h6, P6, Wn6, _row(b6), Wc6a, e0, Wc6b, _row(bc6), Wf, _row(bf))


def kernel(x, edge_index, t, W0, Ws1, Wn1, Ws2, Wn2, Ws3, Wn3, Wt, Ws4, Wn4,
           Wc4, Ws5, Wn5, Wc5, Ws6, Wn6, Wc6, Wf, b0, b1, b2, b3, bt, b4,
           bc4, b5, bc5, b6, bc6, bf):
  srcr = edge_index[0].reshape(NS * CPT, CH)
  dstr = edge_index[1].reshape(NS * CPT, CH)

  zeros = {dh: jnp.zeros((NPAD, dh), jnp.float32) for dh in (16, 32, 48, 64)}
  segk = {dd: _make_segsum(dd) for dd in (32, 64, 96, 128)}

  def seg(g, dd):
    dh = dd // 2
    return segk[dd](g[:, :dh], g[:, dh:], srcr, dstr, zeros[dh])

  half = 128 // 2
  freqs = jnp.exp(-math.log(10000.0)
                  * jnp.arange(half, dtype=jnp.float32) / (half - 1))
  a = jnp.float32(t) * freqs
  temb = jnp.concatenate([jnp.sin(a), jnp.cos(a)])
  temb_p = jnp.zeros((8, 128), jnp.float32).at[0].set(temb)

  def z(dd):
    return jnp.zeros((dd,), jnp.float32)

  e0 = _tc_fused(x, W0, b0)
  p1 = seg(e0, 32)
  h1 = _tc_fused(e0, Ws1, z(64), relu=False)
  e1 = _tc_fused(h1, None, b1, P=p1, Pw=Wn1)
  p2 = seg(e1, 64)
  h2 = _tc_fused(e1, Ws2, z(128), relu=False)
  e2 = _tc_fused(h2, None, b2, P=p2, Pw=Wn2)
  p3 = seg(e2, 128)
  h3 = _tc_fused(e2, Ws3, z(256), relu=False)
  e3, g4 = _post3_g4(h3, p3, Wn3, b3, temb_p, Wt, bt, Wn4)
  p4 = seg(g4, 128)
  h4 = _tc_fused(e3, Ws4, z(128), relu=False)
  d4, g5 = _post4_d4c_g5(h4, p4, b4, Wc4[:128], e2, Wc4[128:], bc4, Wn5)
  p5 = seg(g5, 96)
  h5 = _tc_fused(d4, Ws5, z(96), relu=False)
  d5 = _post5_d5c(h5, p5, b5, Wc5[:96], e1, Wc5[96:], bc5)
  p6 = seg(d5, 96)
  h6 = _tc_fused(d5, Ws6, z(96), relu=False)
  return _post6_final(h6, p6, Wn6, b6, Wc6[:96], e0, Wc6[96:], bc6, Wf, bf)


# speedup vs baseline: 1.1398x; 1.1398x over previous
import functools
import math

import jax
import jax.numpy as jnp
from jax import lax
from jax.experimental import pallas as pl
from jax.experimental.pallas import tpu as pltpu
from jax.experimental.pallas import tpu_sc as plsc

N = 10000
E = 160000
NC = 2
NS = 16
NW = NC * NS
CH = 125
CPW = E // (NW * CH)
NPAD = 10240
RPT = NPAD // NS


def _make_segsum(d):
  mesh = plsc.VectorSubcoreMesh(core_axis_name="c", subcore_axis_name="s")
  nbuf = 2 if d == 128 else 4

  @functools.partial(
      pl.kernel,
      mesh=mesh,
      compiler_params=pltpu.CompilerParams(use_tc_tiling_on_sc=(d == 128)),
      out_type=jax.ShapeDtypeStruct((NC, N, d), jnp.float32),
      scratch_types=[
          pltpu.VMEM((CPW, CH), jnp.int32),
          pltpu.VMEM((CPW, CH), jnp.int32),
          pltpu.VMEM((nbuf, CH, d), jnp.float32),
          pltpu.VMEM_SHARED((NPAD, d), jnp.float32),
          pltpu.SemaphoreType.DMA,
      ],
  )
  def seg(g_hbm, srcr_hbm, dstr_hbm, zeros_hbm, out_hbm,
          src_v, dst_v, rows_v, acc_sh, sem):
    c = lax.axis_index("c")
    s = lax.axis_index("s")
    w = s * NC + c
    zc = pltpu.async_copy(zeros_hbm.at[pl.ds(s * RPT, RPT)],
                          acc_sh.at[pl.ds(s * RPT, RPT)], sem)
    ic = pltpu.async_copy(srcr_hbm.at[pl.ds(w * CPW, CPW)], src_v, sem)
    jc = pltpu.async_copy(dstr_hbm.at[pl.ds(w * CPW, CPW)], dst_v, sem)
    zc.wait()
    ic.wait()
    jc.wait()
    plsc.subcore_barrier()

    for k in range(nbuf - 1):
      pltpu.async_copy(g_hbm.at[src_v.at[k]], rows_v.at[k], sem)

    def body(j, carry):
      @pl.when(j + nbuf - 1 < CPW)
      def _():
        pltpu.async_copy(g_hbm.at[src_v.at[j + nbuf - 1]],
                         rows_v.at[(j + nbuf - 1) % nbuf], sem)
      pltpu.make_async_copy(g_hbm.at[src_v.at[j]],
                            rows_v.at[j % nbuf], sem).wait()
      pltpu.sync_copy(rows_v.at[j % nbuf], acc_sh.at[dst_v.at[j]], add=True)
      return carry

    lax.fori_loop(0, CPW, body, 0)
    plsc.subcore_barrier()

    @pl.when(s < NS - 1)
    def _():
      pltpu.sync_copy(acc_sh.at[pl.ds(s * RPT, RPT)],
                      out_hbm.at[c, pl.ds(s * RPT, RPT)])

    @pl.when(s == NS - 1)
    def _():
      pltpu.sync_copy(acc_sh.at[pl.ds((NS - 1) * RPT, N - (NS - 1) * RPT)],
                      out_hbm.at[c, pl.ds((NS - 1) * RPT, N - (NS - 1) * RPT)])

  return seg


_BM = 2000


def _tc_fused(A, Wa, bias, B=None, Wb=None, P=None, Pw=None,
              temb=None, Wt=None, bt=None, relu=True):
  dO = Wa.shape[1] if Wa is not None else A.shape[1]
  grid = (N // _BM,)

  in_specs = [pl.BlockSpec((_BM, A.shape[1]), lambda i: (i, 0))]
  args = [A]
  if Wa is not None:
    in_specs += [pl.BlockSpec(Wa.shape, lambda i: (0, 0))]
    args += [Wa]
  in_specs += [pl.BlockSpec((1, dO), lambda i: (0, 0))]
  args += [bias.reshape(1, dO)]
  if B is not None:
    in_specs += [pl.BlockSpec((_BM, B.shape[1]), lambda i: (i, 0)),
                 pl.BlockSpec(Wb.shape, lambda i: (0, 0))]
    args += [B, Wb]
  if P is not None:
    in_specs += [pl.BlockSpec((2, _BM, P.shape[2]), lambda i: (0, i, 0))]
    args += [P]
    if Pw is not None:
      in_specs += [pl.BlockSpec(Pw.shape, lambda i: (0, 0))]
      args += [Pw]
  if temb is not None:
    in_specs += [pl.BlockSpec(temb.shape, lambda i: (0, 0)),
                 pl.BlockSpec(Wt.shape, lambda i: (0, 0)),
                 pl.BlockSpec((1, dO), lambda i: (0, 0))]
    args += [temb, Wt, bt.reshape(1, dO)]

  has_Wa = Wa is not None
  has_B = B is not None
  has_P = P is not None
  has_Pw = Pw is not None
  has_t = temb is not None

  def body(*refs):
    it = iter(refs)
    a_ref = next(it)
    if has_Wa:
      wa_ref = next(it)
    b_ref = next(it)
    if has_Wa:
      acc = jnp.dot(a_ref[...], wa_ref[...],
                    preferred_element_type=jnp.float32)
    else:
      acc = a_ref[...]
    if has_B:
      bm_ref = next(it); wb_ref = next(it)
      acc += jnp.dot(bm_ref[...], wb_ref[...],
                     preferred_element_type=jnp.float32)
    if has_P:
      p_ref = next(it)
      psum = p_ref[0] + p_ref[1]
      if has_Pw:
        pw_ref = next(it)
        acc += jnp.dot(psum, pw_ref[...], preferred_element_type=jnp.float32)
      else:
        acc += psum
    acc += b_ref[...]
    if relu:
      acc = jnp.maximum(acc, 0.0)
    if has_t:
      t_ref = next(it); wt_ref = next(it); bt_ref = next(it)
      tv = jnp.dot(t_ref[...], wt_ref[...],
                   preferred_element_type=jnp.float32)
      acc += tv[0:1, :] + bt_ref[...]
    out_ref = next(it)
    out_ref[...] = acc

  return pl.pallas_call(
      body,
      grid=grid,
      in_specs=in_specs,
      out_specs=pl.BlockSpec((_BM, dO), lambda i: (i, 0)),
      out_shape=jax.ShapeDtypeStruct((N, dO), jnp.float32),
  )(*args)


def _row(v):
  return v.reshape(1, -1)


def _psum(p_ref):
  return p_ref[0] + p_ref[1]


def _wspec(W):
  return pl.BlockSpec(W.shape, lambda i: tuple(0 for _ in W.shape))


def _mspec(arr):
  return pl.BlockSpec((_BM, arr.shape[1]), lambda i: (i, 0))


def _pspec(P):
  return pl.BlockSpec((2, _BM, P.shape[2]), lambda i: (0, i, 0))


def _oshape(dO):
  return jax.ShapeDtypeStruct((N, dO), jnp.float32)


def _post3_g4(h3, P, Wn3, b3, temb_p, Wt, bt, Wn4):
  def body(h_ref, p_ref, wn3_ref, b3_ref, t_ref, wt_ref, bt_ref, wn4_ref,
           e3_ref, g4_ref):
    acc = h_ref[...] + jnp.dot(_psum(p_ref), wn3_ref[...],
                               preferred_element_type=jnp.float32)
    acc = jnp.maximum(acc + b3_ref[...], 0.0)
    tv = jnp.dot(t_ref[...], wt_ref[...], preferred_element_type=jnp.float32)
    e3 = acc + tv[0:1, :] + bt_ref[...]
    e3_ref[...] = e3
    g4_ref[...] = jnp.dot(e3, wn4_ref[...], preferred_element_type=jnp.float32)

  return pl.pallas_call(
      body, grid=(N // _BM,),
      in_specs=[_mspec(h3), _pspec(P), _wspec(Wn3), _wspec(_row(b3)),
                _wspec(temb_p), _wspec(Wt), _wspec(_row(bt)), _wspec(Wn4)],
      out_specs=[_mspec(h3), pl.BlockSpec((_BM, 128), lambda i: (i, 0))],
      out_shape=[_oshape(256), _oshape(128)],
  )(h3, P, Wn3, _row(b3), temb_p, Wt, _row(bt), Wn4)


def _post4_d4c_g5(h4, P4, b4, Wc4a, e2, Wc4b, bc4, Wn5):
  def body(h_ref, p_ref, b4_ref, wa_ref, e2_ref, wb_ref, bc_ref, wn5_ref,
           d4_ref, g5_ref):
    tmp = jnp.maximum(h_ref[...] + _psum(p_ref) + b4_ref[...], 0.0)
    acc = (jnp.dot(tmp, wa_ref[...], preferred_element_type=jnp.float32)
           + jnp.dot(e2_ref[...], wb_ref[...],
                     preferred_element_type=jnp.float32))
    d4c = jnp.maximum(acc + bc_ref[...], 0.0)
    d4_ref[...] = d4c
    g5_ref[...] = jnp.dot(d4c, wn5_ref[...],
                          preferred_element_type=jnp.float32)

  return pl.pallas_call(
      body, grid=(N // _BM,),
      in_specs=[_mspec(h4), _pspec(P4), _wspec(_row(b4)), _wspec(Wc4a),
                _mspec(e2), _wspec(Wc4b), _wspec(_row(bc4)), _wspec(Wn5)],
      out_specs=[_mspec(h4), pl.BlockSpec((_BM, 96), lambda i: (i, 0))],
      out_shape=[_oshape(128), _oshape(96)],
  )(h4, P4, _row(b4), Wc4a, e2, Wc4b, _row(bc4), Wn5)


def _post5_d5c(h5, P5, b5, Wc5a, e1, Wc5b, bc5):
  def body(h_ref, p_ref, b5_ref, wa_ref, e1_ref, wb_ref, bc_ref, d5_ref):
    tmp = jnp.maximum(h_ref[...] + _psum(p_ref) + b5_ref[...], 0.0)
    acc = (jnp.dot(tmp, wa_ref[...], preferred_element_type=jnp.float32)
           + jnp.dot(e1_ref[...], wb_ref[...],
                     preferred_element_type=jnp.float32))
    d5_ref[...] = jnp.maximum(acc + bc_ref[...], 0.0)

  return pl.pallas_call(
      body, grid=(N // _BM,),
      in_specs=[_mspec(h5), _pspec(P5), _wspec(_row(b5)), _wspec(Wc5a),
                _mspec(e1), _wspec(Wc5b), _wspec(_row(bc5))],
      out_specs=_mspec(h5),
      out_shape=_oshape(96),
  )(h5, P5, _row(b5), Wc5a, e1, Wc5b, _row(bc5))


def _post6_final(h6, P6, Wn6, b6, Wc6a, e0, Wc6b, bc6, Wf, bf):
  def body(h_ref, p_ref, wn6_ref, b6_ref, wa_ref, e0_ref, wb_ref, bc_ref,
           wf_ref, bf_ref, o_ref):
    tmp = h_ref[...] + jnp.dot(_psum(p_ref), wn6_ref[...],
                               preferred_element_type=jnp.float32)
    tmp = jnp.maximum(tmp + b6_ref[...], 0.0)
    acc = (jnp.dot(tmp, wa_ref[...], preferred_element_type=jnp.float32)
           + jnp.dot(e0_ref[...], wb_ref[...],
                     preferred_element_type=jnp.float32))
    u = jnp.maximum(acc + bc_ref[...], 0.0)
    o_ref[...] = jnp.dot(u, wf_ref[...],
                         preferred_element_type=jnp.float32) + bf_ref[...]

  return pl.pallas_call(
      body, grid=(N // _BM,),
      in_specs=[_mspec(h6), _pspec(P6), _wspec(Wn6), _wspec(_row(b6)),
                _wspec(Wc6a), _mspec(e0), _wspec(Wc6b), _wspec(_row(bc6)),
                _wspec(Wf), _wspec(_row(bf))],
      out_specs=pl.BlockSpec((_BM, 256), lambda i: (i, 0)),
      out_shape=_oshape(256),
  )(h6, P6, Wn6, _row(b6), Wc6a, e0, Wc6b, _row(bc6), Wf, _row(bf))


def kernel(x, edge_index, t, W0, Ws1, Wn1, Ws2, Wn2, Ws3, Wn3, Wt, Ws4, Wn4,
           Wc4, Ws5, Wn5, Wc5, Ws6, Wn6, Wc6, Wf, b0, b1, b2, b3, bt, b4,
           bc4, b5, bc5, b6, bc6, bf):
  srcr = edge_index[0].reshape(NW * CPW, CH)
  dstr = edge_index[1].reshape(NW * CPW, CH)

  zeros = {dd: jnp.zeros((NPAD, dd), jnp.float32) for dd in (32, 64, 96, 128)}
  segk = {dd: _make_segsum(dd) for dd in (32, 64, 96, 128)}

  def seg(g, dd):
    return segk[dd](g, srcr, dstr, zeros[dd])

  half = 128 // 2
  freqs = jnp.exp(-math.log(10000.0)
                  * jnp.arange(half, dtype=jnp.float32) / (half - 1))
  a = jnp.float32(t) * freqs
  temb = jnp.concatenate([jnp.sin(a), jnp.cos(a)])
  temb_p = jnp.zeros((8, 128), jnp.float32).at[0].set(temb)

  def z(dd):
    return jnp.zeros((dd,), jnp.float32)

  e0 = _tc_fused(x, W0, b0)
  p1 = seg(e0, 32)
  h1 = _tc_fused(e0, Ws1, z(64), relu=False)
  e1 = _tc_fused(h1, None, b1, P=p1, Pw=Wn1)
  p2 = seg(e1, 64)
  h2 = _tc_fused(e1, Ws2, z(128), relu=False)
  e2 = _tc_fused(h2, None, b2, P=p2, Pw=Wn2)
  p3 = seg(e2, 128)
  h3 = _tc_fused(e2, Ws3, z(256), relu=False)
  e3, g4 = _post3_g4(h3, p3, Wn3, b3, temb_p, Wt, bt, Wn4)
  p4 = seg(g4, 128)
  h4 = _tc_fused(e3, Ws4, z(128), relu=False)
  d4, g5 = _post4_d4c_g5(h4, p4, b4, Wc4[:128], e2, Wc4[128:], bc4, Wn5)
  p5 = seg(g5, 96)
  h5 = _tc_fused(d4, Ws5, z(96), relu=False)
  d5 = _post5_d5c(h5, p5, b5, Wc5[:96], e1, Wc5[96:], bc5)
  p6 = seg(d5, 96)
  h6 = _tc_fused(d5, Ws6, z(96), relu=False)
  return _post6_final(h6, p6, Wn6, b6, Wc6[:96], e0, Wc6[96:], bc6, Wf, bf)
